# baseline (speedup 1.0000x reference)
import functools

import jax
import jax.numpy as jnp
from jax import lax
from jax.experimental import pallas as pl
from jax.experimental.pallas import tpu as pltpu
from jax.experimental.pallas import tpu_sc as plsc

_CHUNK = 800
_N_BATCH_CHUNKS = 1


def _sc_info():
    info = plsc.get_sparse_core_info()
    return info.num_cores, info.num_subcores


_T_COLS = 2048


@functools.lru_cache(maxsize=None)
def _make_transpose(vocab: int, d: int):
    grid = (vocab + _T_COLS - 1) // _T_COLS

    def transpose_body(t_ref, out_ref):
        y = t_ref[...].T.reshape(_T_COLS * d // 128, 128 // d, d)
        for k in range(128 // d):
            out_ref[:, d * k:d * (k + 1)] = y[:, k, :]

    return pl.pallas_call(
        transpose_body,
        grid=(grid,),
        in_specs=[pl.BlockSpec((d, _T_COLS), lambda i: (0, i))],
        out_specs=pl.BlockSpec((_T_COLS * d // 128, 128), lambda i: (i, 0)),
        out_shape=jax.ShapeDtypeStruct((vocab * d // 128, 128), jnp.float32),
    )


@functools.lru_cache(maxsize=None)
def _make_gather(n_idx: int, d: int):
    nc, ns = _sc_info()
    nw = nc * ns
    assert n_idx % nw == 0
    per_w = n_idx // nw
    assert per_w % (2 * _CHUNK) == 0
    half = per_w // _CHUNK // 2

    mesh = plsc.VectorSubcoreMesh(core_axis_name="c", subcore_axis_name="s")

    @functools.partial(
        pl.kernel,
        mesh=mesh,
        out_type=jax.ShapeDtypeStruct((n_idx, d), jnp.float32),
        scratch_types=[
            pltpu.VMEM((_CHUNK,), jnp.int32),
            pltpu.VMEM((_CHUNK,), jnp.int32),
            pltpu.VMEM((_CHUNK, d), jnp.float32),
            pltpu.VMEM((_CHUNK, d), jnp.float32),
            pltpu.SemaphoreType.DMA,
            pltpu.SemaphoreType.DMA,
            pltpu.SemaphoreType.DMA,
            pltpu.SemaphoreType.DMA,
            pltpu.SemaphoreType.DMA,
            pltpu.SemaphoreType.DMA,
        ],
        compiler_params=pltpu.CompilerParams(
            use_tc_tiling_on_sc=False, needs_layout_passes=False),
    )
    def gather_kernel(idx_hbm, table_hbm, out_hbm, idx0, idx1, rows0, rows1,
                      si0, si1, sg0, sg1, ss0, ss1):
        wid = lax.axis_index("s") * nc + lax.axis_index("c")
        base = wid * per_w

        def idx_load(g, buf, sem):
            pltpu.async_copy(idx_hbm.at[pl.ds(base + g * _CHUNK, _CHUNK)],
                             buf, sem)

        def store(g, buf, sem):
            pltpu.async_copy(buf, out_hbm.at[pl.ds(base + g * _CHUNK, _CHUNK)],
                             sem)

        idx_load(0, idx0, si0)
        idx_load(1, idx1, si1)
        pltpu.make_async_copy(
            idx_hbm.at[pl.ds(base, _CHUNK)], idx0, si0).wait()
        pltpu.async_copy(table_hbm.at[idx0], rows0, sg0)

        def body(t, carry):
            @pl.when(t >= 1)
            def _():
                pltpu.make_async_copy(
                    rows1, out_hbm.at[pl.ds(base, _CHUNK)], ss1).wait()
            pltpu.make_async_copy(
                idx_hbm.at[pl.ds(base, _CHUNK)], idx1, si1).wait()
            pltpu.async_copy(table_hbm.at[idx1], rows1, sg1)
            pltpu.make_async_copy(table_hbm.at[idx0], rows0, sg0).wait()
            store(2 * t, rows0, ss0)

            @pl.when(t < half - 1)
            def _():
                idx_load(2 * t + 2, idx0, si0)

            @pl.when(t < half - 1)
            def _():
                pltpu.make_async_copy(
                    rows0, out_hbm.at[pl.ds(base, _CHUNK)], ss0).wait()
                pltpu.make_async_copy(
                    idx_hbm.at[pl.ds(base, _CHUNK)], idx0, si0).wait()
                pltpu.async_copy(table_hbm.at[idx0], rows0, sg0)
            pltpu.make_async_copy(table_hbm.at[idx1], rows1, sg1).wait()
            store(2 * t + 1, rows1, ss1)

            @pl.when(t < half - 1)
            def _():
                idx_load(2 * t + 3, idx1, si1)

            return carry

        lax.fori_loop(0, half, body, 0)

        pltpu.make_async_copy(rows0, out_hbm.at[pl.ds(base, _CHUNK)], ss0).wait()
        pltpu.make_async_copy(rows1, out_hbm.at[pl.ds(base, _CHUNK)], ss1).wait()

    return gather_kernel


def kernel(x, table):
    b, l = x.shape
    vocab, d = table.shape
    t128 = _make_transpose(vocab, d)(table.T)
    t_lin = t128.reshape(vocab, d)
    cb = b // _N_BATCH_CHUNKS
    gather = _make_gather(cb * l, d)
    outs = []
    for i in range(_N_BATCH_CHUNKS):
        xi = x[i * cb:(i + 1) * cb].reshape(-1).astype(jnp.int32)
        oi = gather(xi, t_lin)
        outs.append(oi.reshape(cb, l * d))
    return jnp.concatenate(outs, axis=0)


# speedup vs baseline: 1.2217x; 1.2217x over previous
import functools

import jax
import jax.numpy as jnp
from jax import lax
from jax.experimental import pallas as pl
from jax.experimental.pallas import tpu as pltpu
from jax.experimental.pallas import tpu_sc as plsc

_CHUNK = 800
_N_BATCH_CHUNKS = 1


def _sc_info():
    info = plsc.get_sparse_core_info()
    return info.num_cores, info.num_subcores


_T_COLS = 8192


@functools.lru_cache(maxsize=None)
def _make_transpose(vocab: int, d: int):
    grid = (vocab + _T_COLS - 1) // _T_COLS

    def transpose_body(t_ref, out_ref):
        out_ref[:, 0:d] = t_ref[...].T

    return pl.pallas_call(
        transpose_body,
        grid=(grid,),
        in_specs=[pl.BlockSpec((d, _T_COLS), lambda i: (0, i))],
        out_specs=pl.BlockSpec((_T_COLS, 128), lambda i: (i, 0)),
        out_shape=jax.ShapeDtypeStruct((vocab, 128), jnp.float32),
    )


@functools.lru_cache(maxsize=None)
def _make_gather(n_idx: int, d: int):
    nc, ns = _sc_info()
    nw = nc * ns
    assert n_idx % nw == 0
    per_w = n_idx // nw
    assert per_w % (2 * _CHUNK) == 0
    half = per_w // _CHUNK // 2

    mesh = plsc.VectorSubcoreMesh(core_axis_name="c", subcore_axis_name="s")

    @functools.partial(
        pl.kernel,
        mesh=mesh,
        out_type=jax.ShapeDtypeStruct((n_idx, d), jnp.float32),
        scratch_types=[
            pltpu.VMEM((_CHUNK,), jnp.int32),
            pltpu.VMEM((_CHUNK,), jnp.int32),
            pltpu.VMEM((_CHUNK, d), jnp.float32),
            pltpu.VMEM((_CHUNK, d), jnp.float32),
            pltpu.SemaphoreType.DMA,
            pltpu.SemaphoreType.DMA,
            pltpu.SemaphoreType.DMA,
            pltpu.SemaphoreType.DMA,
            pltpu.SemaphoreType.DMA,
            pltpu.SemaphoreType.DMA,
        ],
        compiler_params=pltpu.CompilerParams(
            use_tc_tiling_on_sc=False, needs_layout_passes=False),
    )
    def gather_kernel(idx_hbm, table_hbm, out_hbm, idx0, idx1, rows0, rows1,
                      si0, si1, sg0, sg1, ss0, ss1):
        wid = lax.axis_index("s") * nc + lax.axis_index("c")
        base = wid * per_w

        def idx_load(g, buf, sem):
            pltpu.async_copy(idx_hbm.at[pl.ds(base + g * _CHUNK, _CHUNK)],
                             buf, sem)

        def store(g, buf, sem):
            pltpu.async_copy(buf, out_hbm.at[pl.ds(base + g * _CHUNK, _CHUNK)],
                             sem)

        idx_load(0, idx0, si0)
        idx_load(1, idx1, si1)
        pltpu.make_async_copy(
            idx_hbm.at[pl.ds(base, _CHUNK)], idx0, si0).wait()
        pltpu.async_copy(table_hbm.at[idx0], rows0, sg0)

        def body(t, carry):
            @pl.when(t >= 1)
            def _():
                pltpu.make_async_copy(
                    rows1, out_hbm.at[pl.ds(base, _CHUNK)], ss1).wait()
            pltpu.make_async_copy(
                idx_hbm.at[pl.ds(base, _CHUNK)], idx1, si1).wait()
            pltpu.async_copy(table_hbm.at[idx1], rows1, sg1)
            pltpu.make_async_copy(table_hbm.at[idx0], rows0, sg0).wait()
            store(2 * t, rows0, ss0)

            @pl.when(t < half - 1)
            def _():
                idx_load(2 * t + 2, idx0, si0)

            @pl.when(t < half - 1)
            def _():
                pltpu.make_async_copy(
                    rows0, out_hbm.at[pl.ds(base, _CHUNK)], ss0).wait()
                pltpu.make_async_copy(
                    idx_hbm.at[pl.ds(base, _CHUNK)], idx0, si0).wait()
                pltpu.async_copy(table_hbm.at[idx0], rows0, sg0)
            pltpu.make_async_copy(table_hbm.at[idx1], rows1, sg1).wait()
            store(2 * t + 1, rows1, ss1)

            @pl.when(t < half - 1)
            def _():
                idx_load(2 * t + 3, idx1, si1)

            return carry

        lax.fori_loop(0, half, body, 0)

        pltpu.make_async_copy(rows0, out_hbm.at[pl.ds(base, _CHUNK)], ss0).wait()
        pltpu.make_async_copy(rows1, out_hbm.at[pl.ds(base, _CHUNK)], ss1).wait()

    return gather_kernel


def kernel(x, table):
    b, l = x.shape
    vocab, d = table.shape
    tpad = _make_transpose(vocab, d)(table.T)
    t_lin = tpad.reshape(vocab * 128 // d, d)
    cb = b // _N_BATCH_CHUNKS
    gather = _make_gather(cb * l, d)
    outs = []
    for i in range(_N_BATCH_CHUNKS):
        xi = x[i * cb:(i + 1) * cb].reshape(-1).astype(jnp.int32) * (128 // d)
        oi = gather(xi, t_lin)
        outs.append(oi.reshape(cb, l * d))
    return jnp.concatenate(outs, axis=0)


# speedup vs baseline: 1.2707x; 1.0402x over previous
import functools

import jax
import jax.numpy as jnp
from jax import lax
from jax.experimental import pallas as pl
from jax.experimental.pallas import tpu as pltpu
from jax.experimental.pallas import tpu_sc as plsc

_CHUNK = 1600
_N_BATCH_CHUNKS = 1


def _sc_info():
    info = plsc.get_sparse_core_info()
    return info.num_cores, info.num_subcores


_T_COLS = 16384


@functools.lru_cache(maxsize=None)
def _make_transpose(vocab: int, d: int):
    grid = (vocab + _T_COLS - 1) // _T_COLS

    def transpose_body(t_ref, out_ref):
        out_ref[:, 0:d] = t_ref[...].T

    return pl.pallas_call(
        transpose_body,
        grid=(grid,),
        in_specs=[pl.BlockSpec((d, _T_COLS), lambda i: (0, i))],
        out_specs=pl.BlockSpec((_T_COLS, 128), lambda i: (i, 0)),
        out_shape=jax.ShapeDtypeStruct((vocab, 128), jnp.float32),
    )


@functools.lru_cache(maxsize=None)
def _make_gather(n_idx: int, d: int):
    nc, ns = _sc_info()
    nw = nc * ns
    assert n_idx % nw == 0
    per_w = n_idx // nw
    assert per_w % (2 * _CHUNK) == 0
    half = per_w // _CHUNK // 2

    mesh = plsc.VectorSubcoreMesh(core_axis_name="c", subcore_axis_name="s")

    @functools.partial(
        pl.kernel,
        mesh=mesh,
        out_type=jax.ShapeDtypeStruct((n_idx, d), jnp.float32),
        scratch_types=[
            pltpu.VMEM((_CHUNK,), jnp.int32),
            pltpu.VMEM((_CHUNK,), jnp.int32),
            pltpu.VMEM((_CHUNK, d), jnp.float32),
            pltpu.VMEM((_CHUNK, d), jnp.float32),
            pltpu.SemaphoreType.DMA,
            pltpu.SemaphoreType.DMA,
            pltpu.SemaphoreType.DMA,
            pltpu.SemaphoreType.DMA,
            pltpu.SemaphoreType.DMA,
            pltpu.SemaphoreType.DMA,
        ],
        compiler_params=pltpu.CompilerParams(
            use_tc_tiling_on_sc=False, needs_layout_passes=False),
    )
    def gather_kernel(idx_hbm, table_hbm, out_hbm, idx0, idx1, rows0, rows1,
                      si0, si1, sg0, sg1, ss0, ss1):
        wid = lax.axis_index("s") * nc + lax.axis_index("c")
        base = wid * per_w

        def idx_load(g, buf, sem):
            pltpu.async_copy(idx_hbm.at[pl.ds(base + g * _CHUNK, _CHUNK)],
                             buf, sem)

        def store(g, buf, sem):
            pltpu.async_copy(buf, out_hbm.at[pl.ds(base + g * _CHUNK, _CHUNK)],
                             sem)

        idx_load(0, idx0, si0)
        idx_load(1, idx1, si1)
        pltpu.make_async_copy(
            idx_hbm.at[pl.ds(base, _CHUNK)], idx0, si0).wait()
        pltpu.async_copy(table_hbm.at[idx0], rows0, sg0)

        def body(t, carry):
            @pl.when(t >= 1)
            def _():
                pltpu.make_async_copy(
                    rows1, out_hbm.at[pl.ds(base, _CHUNK)], ss1).wait()
            pltpu.make_async_copy(
                idx_hbm.at[pl.ds(base, _CHUNK)], idx1, si1).wait()
            pltpu.async_copy(table_hbm.at[idx1], rows1, sg1)
            pltpu.make_async_copy(table_hbm.at[idx0], rows0, sg0).wait()
            store(2 * t, rows0, ss0)

            @pl.when(t < half - 1)
            def _():
                idx_load(2 * t + 2, idx0, si0)

            @pl.when(t < half - 1)
            def _():
                pltpu.make_async_copy(
                    rows0, out_hbm.at[pl.ds(base, _CHUNK)], ss0).wait()
                pltpu.make_async_copy(
                    idx_hbm.at[pl.ds(base, _CHUNK)], idx0, si0).wait()
                pltpu.async_copy(table_hbm.at[idx0], rows0, sg0)
            pltpu.make_async_copy(table_hbm.at[idx1], rows1, sg1).wait()
            store(2 * t + 1, rows1, ss1)

            @pl.when(t < half - 1)
            def _():
                idx_load(2 * t + 3, idx1, si1)

            return carry

        lax.fori_loop(0, half, body, 0)

        pltpu.make_async_copy(rows0, out_hbm.at[pl.ds(base, _CHUNK)], ss0).wait()
        pltpu.make_async_copy(rows1, out_hbm.at[pl.ds(base, _CHUNK)], ss1).wait()

    return gather_kernel


def kernel(x, table):
    b, l = x.shape
    vocab, d = table.shape
    tpad = _make_transpose(vocab, d)(table.T)
    t_lin = tpad.reshape(vocab * 128 // d, d)
    cb = b // _N_BATCH_CHUNKS
    gather = _make_gather(cb * l, d)
    outs = []
    for i in range(_N_BATCH_CHUNKS):
        xi = x[i * cb:(i + 1) * cb].reshape(-1).astype(jnp.int32) * (128 // d)
        oi = gather(xi, t_lin)
        outs.append(oi.reshape(cb, l * d))
    return jnp.concatenate(outs, axis=0)
